# baseline (speedup 1.0000x reference)
import functools

import jax
import jax.numpy as jnp
from jax import lax
from jax.experimental import pallas as pl
from jax.experimental.pallas import tpu as pltpu
from jax.experimental.pallas import tpu_sc as plsc

_NC = 2
_NS = 16
_NW = _NC * _NS
_CHUNK = 32
_SLOTS = 4
_CORE1_FRAC = 0.265


def _make_sc_gather_sum(n_rows, d, n_pad, rpw0, rpw1):
    rpw_max = max(rpw0, rpw1)
    mesh = plsc.VectorSubcoreMesh(
        core_axis_name="c", subcore_axis_name="s",
        num_cores=_NC, num_subcores=_NS)

    dw = d // 2

    @functools.partial(
        pl.kernel,
        out_type=jax.ShapeDtypeStruct((n_pad, dw), jnp.int32),
        mesh=mesh,
        compiler_params=pltpu.CompilerParams(needs_layout_passes=False),
        scratch_types=[
            [pltpu.VMEM((rpw_max,), jnp.int32) for _ in range(3)],
            [[pltpu.VMEM((_CHUNK, d), jnp.float32) for _ in range(3)]
             for _ in range(_SLOTS)],
            [pltpu.VMEM((_CHUNK, dw), jnp.int32) for _ in range(_SLOTS)],
            [pltpu.SemaphoreType.DMA for _ in range(_SLOTS)],
            [pltpu.SemaphoreType.DMA for _ in range(_SLOTS)],
        ],
    )
    def sc_gather_sum(out2_hbm, nb0_hbm, nb1_hbm, nb2_hbm, gsum_hbm,
                      idx_all, bufs, stage, sems, st_sems):
        cid = lax.axis_index("c")
        sid = lax.axis_index("s")
        rpw = lax.select(cid == 0, rpw0, rpw1)
        base = lax.select(cid == 0, sid * rpw0, _NS * rpw0 + sid * rpw1)
        n_chunks = rpw // _CHUNK
        nbs = (nb0_hbm, nb1_hbm, nb2_hbm)
        for k in range(3):
            pltpu.sync_copy(nbs[k].at[pl.ds(base, rpw_max)], idx_all[k])

        def drain_store(slot):
            pltpu.make_async_copy(
                stage[slot], gsum_hbm.at[pl.ds(0, _CHUNK)],
                st_sems[slot]).wait()

        def fire(ci, slot):
            for k in range(3):
                idx = idx_all[k].at[pl.ds(ci * _CHUNK, _CHUNK)]
                pltpu.async_copy(out2_hbm.at[idx], bufs[slot][k], sems[slot])

        def drain(slot):
            for k in range(3):
                pltpu.make_async_copy(
                    out2_hbm.at[pl.ds(0, _CHUNK)], bufs[slot][k],
                    sems[slot]).wait()

        half = jnp.int32(32767)
        one = jnp.int32(1)
        himask = jnp.int32(-65536)

        def to_bf16_bits(x):
            xi = plsc.bitcast(x, jnp.int32)
            return xi + half + (lax.shift_right_logical(xi, 16) & one)

        def process(ci, slot):
            b0, b1, b2 = bufs[slot]
            st = stage[slot]

            @pl.when(ci >= _SLOTS)
            def _():
                drain_store(slot)

            def row_body(r, rc):
                for c in range(dw // 16):
                    sl_lo = pl.ds(c * 16, 16)
                    sl_hi = pl.ds(dw + c * 16, 16)
                    lo = b0[r, sl_lo] + b1[r, sl_lo] + b2[r, sl_lo]
                    hi = b0[r, sl_hi] + b1[r, sl_hi] + b2[r, sl_hi]
                    lo_b = lax.shift_right_logical(to_bf16_bits(lo), 16)
                    hi_b = to_bf16_bits(hi) & himask
                    st[r, pl.ds(c * 16, 16)] = lo_b | hi_b
                return rc

            lax.fori_loop(0, _CHUNK, row_body, 0)
            pltpu.async_copy(
                st, gsum_hbm.at[pl.ds(base + ci * _CHUNK, _CHUNK)],
                st_sems[slot])

        for s in range(_SLOTS):
            fire(s, s)

        def group_body(p, carry):
            for s in range(_SLOTS):
                ci = _SLOTS * p + s
                drain(s)
                process(ci, s)

                @pl.when(ci + _SLOTS < n_chunks)
                def _():
                    fire(ci + _SLOTS, s)

            return carry

        lax.fori_loop(0, n_chunks // _SLOTS, group_body, 0)
        for s in range(_SLOTS):
            drain_store(s)

    return sc_gather_sum


def _tc3_body(o1, o2, wc1, wc2, bc, out3):
    out3[...] = (
        jnp.dot(o1[...].astype(jnp.bfloat16), wc1[...],
                preferred_element_type=jnp.float32)
        + jnp.dot(o2[...].astype(jnp.bfloat16), wc2[...],
                  preferred_element_type=jnp.float32)
        + bc[...]
    )


def _tc4_body(o2, gi, wal, wah, ba, out4):
    dw = gi.shape[1]
    gw = gi[...]
    g_lo = jax.lax.bitcast_convert_type(gw << 16, jnp.float32)
    g_hi = jax.lax.bitcast_convert_type(gw & jnp.int32(-65536), jnp.float32)
    o2f = o2[...]
    f_lo = ((o2f[:, :dw] + g_lo) * 0.25).astype(jnp.bfloat16)
    f_hi = ((o2f[:, dw:] + g_hi) * 0.25).astype(jnp.bfloat16)
    out4[...] = (
        jnp.dot(f_lo, wal[...], preferred_element_type=jnp.float32)
        + jnp.dot(f_hi, wah[...], preferred_element_type=jnp.float32)
        + ba[...]
    )


def kernel(out1, out2, neighbour, W_comb, b_comb, W_agg, b_agg):
    n, d = out2.shape
    dout = b_comb.shape[0]

    step = _SLOTS * _CHUNK
    per_pair = ((n + _NS - 1) // _NS + step - 1) // step * step
    rpw1 = max(step, int(round(per_pair * _CORE1_FRAC / step)) * step)
    rpw0 = per_pair - rpw1
    n_pad = _NS * per_pair
    nbt = jnp.transpose(neighbour.astype(jnp.int32))
    nbt = jnp.pad(nbt, ((0, 0), (0, n_pad + max(rpw0, rpw1) - n)))
    gsum = _make_sc_gather_sum(n, d, n_pad, rpw0, rpw1)(
        out2, nbt[0], nbt[1], nbt[2])

    wcT = jnp.transpose(W_comb[:, :, 0]).astype(jnp.bfloat16)
    wc1 = wcT[:d]
    wc2 = wcT[d:]
    wa = jnp.transpose(W_agg[:, :, 0]).astype(jnp.bfloat16)
    bc = b_comb.reshape(1, dout)
    ba = b_agg.reshape(1, dout)

    blk = 2000
    assert n % blk == 0
    grid = (n // blk,)
    row_spec = pl.BlockSpec((blk, d), lambda i: (i, 0))
    out_spec = pl.BlockSpec((blk, dout), lambda i: (i, 0))
    full = lambda s: pl.BlockSpec(s, lambda i: (0, 0))
    out_ty = jax.ShapeDtypeStruct((n, dout), jnp.float32)
    out3 = pl.pallas_call(
        _tc3_body,
        grid=grid,
        in_specs=[row_spec, row_spec, full((d, dout)), full((d, dout)),
                  full((1, dout))],
        out_specs=out_spec,
        out_shape=out_ty,
    )(out1, out2, wc1, wc2, bc)
    dw = d // 2
    out4 = pl.pallas_call(
        _tc4_body,
        grid=grid,
        in_specs=[row_spec, pl.BlockSpec((blk, dw), lambda i: (i, 0)),
                  full((dw, dout)), full((dw, dout)), full((1, dout))],
        out_specs=out_spec,
        out_shape=out_ty,
    )(out2, gsum, wa[:dw], wa[dw:], ba)
    return (out3, out4)


# speedup vs baseline: 1.1407x; 1.1407x over previous
import functools

import jax
import jax.numpy as jnp
from jax import lax
from jax.experimental import pallas as pl
from jax.experimental.pallas import tpu as pltpu
from jax.experimental.pallas import tpu_sc as plsc

_NC = 2
_NS = 16
_NW = _NC * _NS
_CHUNK = 32
_SLOTS = 4
_CORE1_FRAC = 0.265


def _make_sc_gather_sum(n_rows, d, n_pad, rpw0, rpw1):
    rpw_max = max(rpw0, rpw1)
    mesh = plsc.VectorSubcoreMesh(
        core_axis_name="c", subcore_axis_name="s",
        num_cores=_NC, num_subcores=_NS)

    @functools.partial(
        pl.kernel,
        out_type=jax.ShapeDtypeStruct((n_pad, d), jnp.float32),
        mesh=mesh,
        scratch_types=[
            [pltpu.VMEM((rpw_max,), jnp.int32) for _ in range(3)],
            [[pltpu.VMEM((_CHUNK, d), jnp.float32) for _ in range(3)]
             for _ in range(_SLOTS)],
            [pltpu.SemaphoreType.DMA for _ in range(_SLOTS)],
            [pltpu.SemaphoreType.DMA for _ in range(_SLOTS)],
        ],
    )
    def sc_gather_sum(out2_hbm, nb0_hbm, nb1_hbm, nb2_hbm, gsum_hbm,
                      idx_all, bufs, sems, st_sems):
        cid = lax.axis_index("c")
        sid = lax.axis_index("s")
        rpw = lax.select(cid == 0, rpw0, rpw1)
        base = lax.select(cid == 0, sid * rpw0, _NS * rpw0 + sid * rpw1)
        n_chunks = rpw // _CHUNK
        nbs = (nb0_hbm, nb1_hbm, nb2_hbm)
        for k in range(3):
            pltpu.sync_copy(nbs[k].at[pl.ds(base, rpw_max)], idx_all[k])

        def drain_store(slot):
            pltpu.make_async_copy(
                bufs[slot][0], gsum_hbm.at[pl.ds(0, _CHUNK)],
                st_sems[slot]).wait()

        def fire(ci, slot, first=False):
            for k in (1, 2):
                idx = idx_all[k].at[pl.ds(ci * _CHUNK, _CHUNK)]
                pltpu.async_copy(out2_hbm.at[idx], bufs[slot][k], sems[slot])
            if not first:
                drain_store(slot)
            idx = idx_all[0].at[pl.ds(ci * _CHUNK, _CHUNK)]
            pltpu.async_copy(out2_hbm.at[idx], bufs[slot][0], sems[slot])

        def drain(slot):
            for k in range(3):
                pltpu.make_async_copy(
                    out2_hbm.at[pl.ds(0, _CHUNK)], bufs[slot][k],
                    sems[slot]).wait()

        def process(ci, slot):
            b0, b1, b2 = bufs[slot]

            def row_body(r, rc):
                for c in range(d // 16):
                    sl = pl.ds(c * 16, 16)
                    plsc.addupdate(b0.at[r, sl], b1[r, sl] + b2[r, sl])
                return rc

            lax.fori_loop(0, _CHUNK, row_body, 0)
            pltpu.async_copy(
                b0, gsum_hbm.at[pl.ds(base + ci * _CHUNK, _CHUNK)],
                st_sems[slot])

        for s in range(_SLOTS):
            fire(s, s, first=True)

        def group_body(p, carry):
            for s in range(_SLOTS):
                ci = _SLOTS * p + s
                drain(s)
                process(ci, s)

                @pl.when(ci + _SLOTS < n_chunks)
                def _():
                    fire(ci + _SLOTS, s)

            return carry

        lax.fori_loop(0, n_chunks // _SLOTS, group_body, 0)
        for s in range(_SLOTS):
            drain_store(s)

    return sc_gather_sum


def _cast_body(x, out):
    out[...] = x[...].astype(jnp.bfloat16)


def _tc3_body(o1, o2, wc1, wc2, bc, out3):
    out3[...] = (
        jnp.dot(o1[...], wc1[...], preferred_element_type=jnp.float32)
        + jnp.dot(o2[...], wc2[...], preferred_element_type=jnp.float32)
        + bc[...]
    )


def _tc4_body(o2, g, wa, ba, out4):
    f = ((o2[...].astype(jnp.float32) + g[...]) * 0.25).astype(jnp.bfloat16)
    out4[...] = jnp.dot(f, wa[...], preferred_element_type=jnp.float32) + ba[...]


def kernel(out1, out2, neighbour, W_comb, b_comb, W_agg, b_agg):
    n, d = out2.shape
    dout = b_comb.shape[0]

    step = _SLOTS * _CHUNK
    per_pair = ((n + _NS - 1) // _NS + step - 1) // step * step
    rpw1 = max(step, int(round(per_pair * _CORE1_FRAC / step)) * step)
    rpw0 = per_pair - rpw1
    n_pad = _NS * per_pair
    nbt = jnp.transpose(neighbour.astype(jnp.int32))
    nbt = jnp.pad(nbt, ((0, 0), (0, n_pad + max(rpw0, rpw1) - n)))
    gsum = _make_sc_gather_sum(n, d, n_pad, rpw0, rpw1)(
        out2, nbt[0], nbt[1], nbt[2])

    wcT = jnp.transpose(W_comb[:, :, 0]).astype(jnp.bfloat16)
    wc1 = wcT[:d]
    wc2 = wcT[d:]
    wa = jnp.transpose(W_agg[:, :, 0]).astype(jnp.bfloat16)
    bc = b_comb.reshape(1, dout)
    ba = b_agg.reshape(1, dout)

    blk = 2000
    assert n % blk == 0
    grid = (n // blk,)
    row_spec = pl.BlockSpec((blk, d), lambda i: (i, 0))
    out_spec = pl.BlockSpec((blk, dout), lambda i: (i, 0))
    full = lambda s: pl.BlockSpec(s, lambda i: (0, 0))
    out_ty = jax.ShapeDtypeStruct((n, dout), jnp.float32)
    bf_ty = jax.ShapeDtypeStruct((n, d), jnp.bfloat16)

    cast = lambda x: pl.pallas_call(
        _cast_body, grid=grid, in_specs=[row_spec],
        out_specs=row_spec, out_shape=bf_ty)(x)
    out1_bf = cast(out1)
    out2_bf = cast(out2)

    out3 = pl.pallas_call(
        _tc3_body,
        grid=grid,
        in_specs=[row_spec, row_spec, full((d, dout)), full((d, dout)),
                  full((1, dout))],
        out_specs=out_spec,
        out_shape=out_ty,
    )(out1_bf, out2_bf, wc1, wc2, bc)
    out4 = pl.pallas_call(
        _tc4_body,
        grid=grid,
        in_specs=[row_spec, row_spec, full((d, dout)), full((1, dout))],
        out_specs=out_spec,
        out_shape=out_ty,
    )(out2_bf, gsum, wa, ba)
    return (out3, out4)


# speedup vs baseline: 1.2143x; 1.0645x over previous
import functools

import jax
import jax.numpy as jnp
from jax import lax
from jax.experimental import pallas as pl
from jax.experimental.pallas import tpu as pltpu
from jax.experimental.pallas import tpu_sc as plsc

_NC = 2
_NS = 16
_NW = _NC * _NS
_CHUNK = 32
_SLOTS = 4
_CORE1_FRAC = 0.265


def _make_sc_gather_sum(n_rows, d, n_pad, rpw0, rpw1):
    rpw_max = max(rpw0, rpw1)
    mesh = plsc.VectorSubcoreMesh(
        core_axis_name="c", subcore_axis_name="s",
        num_cores=_NC, num_subcores=_NS)

    @functools.partial(
        pl.kernel,
        out_type=jax.ShapeDtypeStruct((n_pad, d), jnp.float32),
        mesh=mesh,
        scratch_types=[
            [pltpu.VMEM((rpw_max,), jnp.int32) for _ in range(3)],
            [[pltpu.VMEM((_CHUNK, d), jnp.float32) for _ in range(3)]
             for _ in range(_SLOTS)],
            [pltpu.SemaphoreType.DMA for _ in range(_SLOTS)],
            [pltpu.SemaphoreType.DMA for _ in range(_SLOTS)],
        ],
    )
    def sc_gather_sum(out2_hbm, nb0_hbm, nb1_hbm, nb2_hbm, gsum_hbm,
                      idx_all, bufs, sems, st_sems):
        cid = lax.axis_index("c")
        sid = lax.axis_index("s")
        rpw = lax.select(cid == 0, rpw0, rpw1)
        base = lax.select(cid == 0, sid * rpw0, _NS * rpw0 + sid * rpw1)
        n_chunks = rpw // _CHUNK
        nbs = (nb0_hbm, nb1_hbm, nb2_hbm)
        for k in range(3):
            pltpu.sync_copy(nbs[k].at[pl.ds(base, rpw_max)], idx_all[k])

        def drain_store(slot):
            pltpu.make_async_copy(
                bufs[slot][0], gsum_hbm.at[pl.ds(0, _CHUNK)],
                st_sems[slot]).wait()

        def fire(ci, slot, first=False):
            for k in (1, 2):
                idx = idx_all[k].at[pl.ds(ci * _CHUNK, _CHUNK)]
                pltpu.async_copy(out2_hbm.at[idx], bufs[slot][k], sems[slot])
            if not first:
                drain_store(slot)
            idx = idx_all[0].at[pl.ds(ci * _CHUNK, _CHUNK)]
            pltpu.async_copy(out2_hbm.at[idx], bufs[slot][0], sems[slot])

        def drain(slot):
            for k in range(3):
                pltpu.make_async_copy(
                    out2_hbm.at[pl.ds(0, _CHUNK)], bufs[slot][k],
                    sems[slot]).wait()

        def process(ci, slot):
            b0, b1, b2 = bufs[slot]

            def row_body(r, rc):
                for c in range(d // 16):
                    sl = pl.ds(c * 16, 16)
                    plsc.addupdate(b0.at[r, sl], b1[r, sl] + b2[r, sl])
                return rc

            lax.fori_loop(0, _CHUNK, row_body, 0)
            pltpu.async_copy(
                b0, gsum_hbm.at[pl.ds(base + ci * _CHUNK, _CHUNK)],
                st_sems[slot])

        for s in range(_SLOTS):
            fire(s, s, first=True)

        def group_body(p, carry):
            for s in range(_SLOTS):
                ci = _SLOTS * p + s
                drain(s)
                process(ci, s)

                @pl.when(ci + _SLOTS < n_chunks)
                def _():
                    fire(ci + _SLOTS, s)

            return carry

        lax.fori_loop(0, n_chunks // _SLOTS, group_body, 0)
        for s in range(_SLOTS):
            drain_store(s)

    return sc_gather_sum


def _cast_body(x, out):
    out[...] = x[...].astype(jnp.bfloat16)


def _tc3_body(o1, o2, wc1, wc2, bc, out3):
    out3[...] = (
        jnp.dot(o1[...].astype(jnp.bfloat16), wc1[...],
                preferred_element_type=jnp.float32)
        + jnp.dot(o2[...], wc2[...], preferred_element_type=jnp.float32)
        + bc[...]
    )


def _tc4_body(o2, g, wa, ba, out4):
    f = ((o2[...].astype(jnp.float32) + g[...]) * 0.25).astype(jnp.bfloat16)
    out4[...] = jnp.dot(f, wa[...], preferred_element_type=jnp.float32) + ba[...]


def kernel(out1, out2, neighbour, W_comb, b_comb, W_agg, b_agg):
    n, d = out2.shape
    dout = b_comb.shape[0]

    step = _SLOTS * _CHUNK
    per_pair = ((n + _NS - 1) // _NS + step - 1) // step * step
    rpw1 = max(step, int(round(per_pair * _CORE1_FRAC / step)) * step)
    rpw0 = per_pair - rpw1
    n_pad = _NS * per_pair
    nbt = jnp.transpose(neighbour.astype(jnp.int32))
    nbt = jnp.pad(nbt, ((0, 0), (0, n_pad + max(rpw0, rpw1) - n)))
    gsum = _make_sc_gather_sum(n, d, n_pad, rpw0, rpw1)(
        out2, nbt[0], nbt[1], nbt[2])

    wcT = jnp.transpose(W_comb[:, :, 0]).astype(jnp.bfloat16)
    wc1 = wcT[:d]
    wc2 = wcT[d:]
    wa = jnp.transpose(W_agg[:, :, 0]).astype(jnp.bfloat16)
    bc = b_comb.reshape(1, dout)
    ba = b_agg.reshape(1, dout)

    blk = 2000
    assert n % blk == 0
    grid = (n // blk,)
    row_spec = pl.BlockSpec((blk, d), lambda i: (i, 0))
    out_spec = pl.BlockSpec((blk, dout), lambda i: (i, 0))
    full = lambda s: pl.BlockSpec(s, lambda i: (0, 0))
    out_ty = jax.ShapeDtypeStruct((n, dout), jnp.float32)
    bf_ty = jax.ShapeDtypeStruct((n, d), jnp.bfloat16)

    cast = lambda x: pl.pallas_call(
        _cast_body, grid=grid, in_specs=[row_spec],
        out_specs=row_spec, out_shape=bf_ty)(x)
    out2_bf = cast(out2)

    out3 = pl.pallas_call(
        _tc3_body,
        grid=grid,
        in_specs=[row_spec, row_spec, full((d, dout)), full((d, dout)),
                  full((1, dout))],
        out_specs=out_spec,
        out_shape=out_ty,
    )(out1, out2_bf, wc1, wc2, bc)
    out4 = pl.pallas_call(
        _tc4_body,
        grid=grid,
        in_specs=[row_spec, row_spec, full((d, dout)), full((1, dout))],
        out_specs=out_spec,
        out_shape=out_ty,
    )(out2_bf, gsum, wa, ba)
    return (out3, out4)


# speedup vs baseline: 1.2389x; 1.0203x over previous
import functools

import jax
import jax.numpy as jnp
from jax import lax
from jax.experimental import pallas as pl
from jax.experimental.pallas import tpu as pltpu
from jax.experimental.pallas import tpu_sc as plsc

_NC = 2
_NS = 16
_NW = _NC * _NS
_CHUNK = 32
_SLOTS = 4
_CORE1_FRAC = 0.265


def _make_sc_gather_sum(n_rows, d, n_pad, rpw0, rpw1):
    rpw_max = max(rpw0, rpw1)
    mesh = plsc.VectorSubcoreMesh(
        core_axis_name="c", subcore_axis_name="s",
        num_cores=_NC, num_subcores=_NS)

    @functools.partial(
        pl.kernel,
        out_type=jax.ShapeDtypeStruct((n_pad, d), jnp.float32),
        mesh=mesh,
        scratch_types=[
            [pltpu.VMEM((rpw_max,), jnp.int32) for _ in range(3)],
            [[pltpu.VMEM((_CHUNK, d), jnp.float32) for _ in range(3)]
             for _ in range(_SLOTS)],
            [pltpu.SemaphoreType.DMA for _ in range(_SLOTS)],
            [pltpu.SemaphoreType.DMA for _ in range(_SLOTS)],
        ],
    )
    def sc_gather_sum(out2_hbm, nb0_hbm, nb1_hbm, nb2_hbm, gsum_hbm,
                      idx_all, bufs, sems, st_sems):
        cid = lax.axis_index("c")
        sid = lax.axis_index("s")
        rpw = lax.select(cid == 0, rpw0, rpw1)
        base = lax.select(cid == 0, sid * rpw0, _NS * rpw0 + sid * rpw1)
        n_chunks = rpw // _CHUNK
        nbs = (nb0_hbm, nb1_hbm, nb2_hbm)
        for k in range(3):
            pltpu.sync_copy(nbs[k].at[pl.ds(base, rpw_max)], idx_all[k])

        def drain_store(slot):
            pltpu.make_async_copy(
                bufs[slot][0], gsum_hbm.at[pl.ds(0, _CHUNK)],
                st_sems[slot]).wait()

        def fire(ci, slot, first=False):
            for k in (1, 2):
                idx = idx_all[k].at[pl.ds(ci * _CHUNK, _CHUNK)]
                pltpu.async_copy(out2_hbm.at[idx], bufs[slot][k], sems[slot])
            if not first:
                drain_store(slot)
            idx = idx_all[0].at[pl.ds(ci * _CHUNK, _CHUNK)]
            pltpu.async_copy(out2_hbm.at[idx], bufs[slot][0], sems[slot])

        def drain(slot):
            for k in range(3):
                pltpu.make_async_copy(
                    out2_hbm.at[pl.ds(0, _CHUNK)], bufs[slot][k],
                    sems[slot]).wait()

        def process(ci, slot):
            b0, b1, b2 = bufs[slot]

            def row_body(r, rc):
                for c in range(d // 16):
                    sl = pl.ds(c * 16, 16)
                    plsc.addupdate(b0.at[r, sl], b1[r, sl] + b2[r, sl])
                return rc

            lax.fori_loop(0, _CHUNK, row_body, 0)
            pltpu.async_copy(
                b0, gsum_hbm.at[pl.ds(base + ci * _CHUNK, _CHUNK)],
                st_sems[slot])

        for s in range(_SLOTS):
            fire(s, s, first=True)

        def group_body(p, carry):
            for s in range(_SLOTS):
                ci = _SLOTS * p + s
                drain(s)
                process(ci, s)

                @pl.when(ci + _SLOTS < n_chunks)
                def _():
                    fire(ci + _SLOTS, s)

            return carry

        lax.fori_loop(0, n_chunks // _SLOTS, group_body, 0)
        for s in range(_SLOTS):
            drain_store(s)

    return sc_gather_sum


def _tc3_body(o1, o2, wc1, wc2, bc, out3):
    out3[...] = (
        jnp.dot(o1[...].astype(jnp.bfloat16), wc1[...],
                preferred_element_type=jnp.float32)
        + jnp.dot(o2[...].astype(jnp.bfloat16), wc2[...],
                  preferred_element_type=jnp.float32)
        + bc[...]
    )


def _tc4_body(o2, g, wa, ba, out4):
    f = ((o2[...] + g[...]) * 0.25).astype(jnp.bfloat16)
    out4[...] = jnp.dot(f, wa[...], preferred_element_type=jnp.float32) + ba[...]


def kernel(out1, out2, neighbour, W_comb, b_comb, W_agg, b_agg):
    n, d = out2.shape
    dout = b_comb.shape[0]

    step = _SLOTS * _CHUNK
    per_pair = ((n + _NS - 1) // _NS + step - 1) // step * step
    rpw1 = max(step, int(round(per_pair * _CORE1_FRAC / step)) * step)
    rpw0 = per_pair - rpw1
    n_pad = _NS * per_pair
    nbt = jnp.transpose(neighbour.astype(jnp.int32))
    nbt = jnp.pad(nbt, ((0, 0), (0, n_pad + max(rpw0, rpw1) - n)))
    gsum = _make_sc_gather_sum(n, d, n_pad, rpw0, rpw1)(
        out2, nbt[0], nbt[1], nbt[2])

    wcT = jnp.transpose(W_comb[:, :, 0]).astype(jnp.bfloat16)
    wc1 = wcT[:d]
    wc2 = wcT[d:]
    wa = jnp.transpose(W_agg[:, :, 0]).astype(jnp.bfloat16)
    bc = b_comb.reshape(1, dout)
    ba = b_agg.reshape(1, dout)

    blk = 2000
    assert n % blk == 0
    grid = (n // blk,)
    row_spec = pl.BlockSpec((blk, d), lambda i: (i, 0))
    out_spec = pl.BlockSpec((blk, dout), lambda i: (i, 0))
    full = lambda s: pl.BlockSpec(s, lambda i: (0, 0))
    out_ty = jax.ShapeDtypeStruct((n, dout), jnp.float32)
    out3 = pl.pallas_call(
        _tc3_body,
        grid=grid,
        in_specs=[row_spec, row_spec, full((d, dout)), full((d, dout)),
                  full((1, dout))],
        out_specs=out_spec,
        out_shape=out_ty,
    )(out1, out2, wc1, wc2, bc)
    out4 = pl.pallas_call(
        _tc4_body,
        grid=grid,
        in_specs=[row_spec, row_spec, full((d, dout)), full((1, dout))],
        out_specs=out_spec,
        out_shape=out_ty,
    )(out2, gsum, wa, ba)
    return (out3, out4)


# speedup vs baseline: 1.2409x; 1.0016x over previous
import functools

import jax
import jax.numpy as jnp
from jax import lax
from jax.experimental import pallas as pl
from jax.experimental.pallas import tpu as pltpu
from jax.experimental.pallas import tpu_sc as plsc

_NC = 2
_NS = 16
_NW = _NC * _NS
_CHUNK = 32
_SLOTS = 4
_CORE1_FRAC = 0.265


def _make_sc_gather_sum(n_rows, d, n_pad, rpw0, rpw1, row_off=0):
    rpw_max = max(rpw0, rpw1)
    mesh = plsc.VectorSubcoreMesh(
        core_axis_name="c", subcore_axis_name="s",
        num_cores=_NC, num_subcores=_NS)

    @functools.partial(
        pl.kernel,
        out_type=jax.ShapeDtypeStruct((n_pad, d), jnp.float32),
        mesh=mesh,
        scratch_types=[
            [pltpu.VMEM((rpw_max,), jnp.int32) for _ in range(3)],
            [[pltpu.VMEM((_CHUNK, d), jnp.float32) for _ in range(3)]
             for _ in range(_SLOTS)],
            [pltpu.SemaphoreType.DMA for _ in range(_SLOTS)],
            [pltpu.SemaphoreType.DMA for _ in range(_SLOTS)],
        ],
    )
    def sc_gather_sum(out2_hbm, nb0_hbm, nb1_hbm, nb2_hbm, gsum_hbm,
                      idx_all, bufs, sems, st_sems):
        cid = lax.axis_index("c")
        sid = lax.axis_index("s")
        rpw = lax.select(cid == 0, rpw0, rpw1)
        base = lax.select(cid == 0, sid * rpw0, _NS * rpw0 + sid * rpw1)
        n_chunks = rpw // _CHUNK
        nbs = (nb0_hbm, nb1_hbm, nb2_hbm)
        for k in range(3):
            pltpu.sync_copy(nbs[k].at[pl.ds(row_off + base, rpw_max)],
                            idx_all[k])

        def drain_store(slot):
            pltpu.make_async_copy(
                bufs[slot][0], gsum_hbm.at[pl.ds(0, _CHUNK)],
                st_sems[slot]).wait()

        def fire(ci, slot, first=False):
            for k in (1, 2):
                idx = idx_all[k].at[pl.ds(ci * _CHUNK, _CHUNK)]
                pltpu.async_copy(out2_hbm.at[idx], bufs[slot][k], sems[slot])
            if not first:
                drain_store(slot)
            idx = idx_all[0].at[pl.ds(ci * _CHUNK, _CHUNK)]
            pltpu.async_copy(out2_hbm.at[idx], bufs[slot][0], sems[slot])

        def drain(slot):
            for k in range(3):
                pltpu.make_async_copy(
                    out2_hbm.at[pl.ds(0, _CHUNK)], bufs[slot][k],
                    sems[slot]).wait()

        def process(ci, slot):
            b0, b1, b2 = bufs[slot]

            def row_body(r, rc):
                for c in range(d // 16):
                    sl = pl.ds(c * 16, 16)
                    plsc.addupdate(b0.at[r, sl], b1[r, sl] + b2[r, sl])
                return rc

            lax.fori_loop(0, _CHUNK, row_body, 0)
            pltpu.async_copy(
                b0, gsum_hbm.at[pl.ds(base + ci * _CHUNK, _CHUNK)],
                st_sems[slot])

        for s in range(_SLOTS):
            fire(s, s, first=True)

        def group_body(p, carry):
            for s in range(_SLOTS):
                ci = _SLOTS * p + s
                drain(s)
                process(ci, s)

                @pl.when(ci + _SLOTS < n_chunks)
                def _():
                    fire(ci + _SLOTS, s)

            return carry

        lax.fori_loop(0, n_chunks // _SLOTS, group_body, 0)
        for s in range(_SLOTS):
            drain_store(s)

    return sc_gather_sum


def _tc3_body(o1, o2, wc1, wc2, bc, out3):
    out3[...] = (
        jnp.dot(o1[...].astype(jnp.bfloat16), wc1[...],
                preferred_element_type=jnp.float32)
        + jnp.dot(o2[...].astype(jnp.bfloat16), wc2[...],
                  preferred_element_type=jnp.float32)
        + bc[...]
    )


def _tc4_body(o2, g, wa, ba, out4):
    f = ((o2[...] + g[...]) * 0.25).astype(jnp.bfloat16)
    out4[...] = jnp.dot(f, wa[...], preferred_element_type=jnp.float32) + ba[...]


def _split_pairs(per_pair):
    step = _SLOTS * _CHUNK
    rpw1 = max(step, int(round(per_pair * _CORE1_FRAC / step)) * step)
    return per_pair - rpw1, rpw1


_STAGE_A_FRAC = 0.69


def kernel(out1, out2, neighbour, W_comb, b_comb, W_agg, b_agg):
    n, d = out2.shape
    dout = b_comb.shape[0]

    step = _SLOTS * _CHUNK
    per_pair = ((n + _NS - 1) // _NS + step - 1) // step * step
    pair_a = max(step, int(round(per_pair * _STAGE_A_FRAC / step)) * step)
    n_a = _NS * pair_a
    pair_b = max(step,
                 ((n - n_a + _NS - 1) // _NS + step - 1) // step * step)
    n_b = _NS * pair_b
    rpw0a, rpw1a = _split_pairs(pair_a)
    rpw0b, rpw1b = _split_pairs(pair_b)
    nbt = jnp.transpose(neighbour.astype(jnp.int32))
    nbt = jnp.pad(
        nbt,
        ((0, 0), (0, n_a + n_b + max(rpw0a, rpw1a, rpw0b, rpw1b) - n)))
    gsum_a = _make_sc_gather_sum(n, d, n_a, rpw0a, rpw1a)(
        out2, nbt[0], nbt[1], nbt[2])
    gsum_b = _make_sc_gather_sum(n, d, n_b, rpw0b, rpw1b, row_off=n_a)(
        out2, nbt[0], nbt[1], nbt[2])

    wcT = jnp.transpose(W_comb[:, :, 0]).astype(jnp.bfloat16)
    wc1 = wcT[:d]
    wc2 = wcT[d:]
    wa = jnp.transpose(W_agg[:, :, 0]).astype(jnp.bfloat16)
    bc = b_comb.reshape(1, dout)
    ba = b_agg.reshape(1, dout)

    blk = 2048
    assert n_a % blk == 0
    row_spec = pl.BlockSpec((blk, d), lambda i: (i, 0))
    out_spec = pl.BlockSpec((blk, dout), lambda i: (i, 0))
    full = lambda s: pl.BlockSpec(s, lambda i: (0, 0))
    out_ty = jax.ShapeDtypeStruct((n, dout), jnp.float32)

    out3 = pl.pallas_call(
        _tc3_body,
        grid=(-(-n // blk),),
        in_specs=[row_spec, row_spec, full((d, dout)), full((d, dout)),
                  full((1, dout))],
        out_specs=out_spec,
        out_shape=out_ty,
    )(out1, out2, wc1, wc2, bc)

    out4_a = pl.pallas_call(
        _tc4_body,
        grid=(n_a // blk,),
        in_specs=[row_spec, row_spec, full((d, dout)), full((1, dout))],
        out_specs=out_spec,
        out_shape=out_ty,
    )(out2, gsum_a, wa, ba)

    nblk_b = -(-n // blk) - n_a // blk
    off_blk = n_a // blk

    def _tc4b_body(prev, o2, g, wa_, ba_, out4):
        del prev
        _tc4_body(o2, g, wa_, ba_, out4)

    out4 = pl.pallas_call(
        _tc4b_body,
        grid=(nblk_b,),
        in_specs=[
            pl.BlockSpec(memory_space=pl.ANY),
            pl.BlockSpec((blk, d), lambda i, o=off_blk: (i + o, 0)),
            pl.BlockSpec((blk, d), lambda i: (i, 0)),
            full((d, dout)), full((1, dout)),
        ],
        out_specs=pl.BlockSpec((blk, dout), lambda i, o=off_blk: (i + o, 0)),
        out_shape=out_ty,
        input_output_aliases={0: 0},
    )(out4_a, out2, gsum_b, wa, ba)
    return (out3, out4)


# speedup vs baseline: 1.2423x; 1.0011x over previous
import functools

import jax
import jax.numpy as jnp
from jax import lax
from jax.experimental import pallas as pl
from jax.experimental.pallas import tpu as pltpu
from jax.experimental.pallas import tpu_sc as plsc

_NC = 2
_NS = 16
_NW = _NC * _NS
_CHUNK = 32
_SLOTS = 4
_CORE1_FRAC = 0.265


def _make_sc_gather_sum(n_rows, d, n_pad, rpw0, rpw1, row_off=0):
    rpw_max = max(rpw0, rpw1)
    mesh = plsc.VectorSubcoreMesh(
        core_axis_name="c", subcore_axis_name="s",
        num_cores=_NC, num_subcores=_NS)

    @functools.partial(
        pl.kernel,
        out_type=jax.ShapeDtypeStruct((n_pad, d), jnp.float32),
        mesh=mesh,
        scratch_types=[
            [pltpu.VMEM((rpw_max,), jnp.int32) for _ in range(3)],
            [[pltpu.VMEM((_CHUNK, d), jnp.float32) for _ in range(3)]
             for _ in range(_SLOTS)],
            [pltpu.SemaphoreType.DMA for _ in range(_SLOTS)],
            [pltpu.SemaphoreType.DMA for _ in range(_SLOTS)],
        ],
    )
    def sc_gather_sum(out2_hbm, nb0_hbm, nb1_hbm, nb2_hbm, gsum_hbm,
                      idx_all, bufs, sems, st_sems):
        cid = lax.axis_index("c")
        sid = lax.axis_index("s")
        rpw = lax.select(cid == 0, rpw0, rpw1)
        base = lax.select(cid == 0, sid * rpw0, _NS * rpw0 + sid * rpw1)
        n_chunks = rpw // _CHUNK
        nbs = (nb0_hbm, nb1_hbm, nb2_hbm)
        for k in range(3):
            pltpu.sync_copy(nbs[k].at[pl.ds(row_off + base, rpw_max)],
                            idx_all[k])

        def drain_store(slot):
            pltpu.make_async_copy(
                bufs[slot][0], gsum_hbm.at[pl.ds(0, _CHUNK)],
                st_sems[slot]).wait()

        def fire(ci, slot, first=False):
            for k in (1, 2):
                idx = idx_all[k].at[pl.ds(ci * _CHUNK, _CHUNK)]
                pltpu.async_copy(out2_hbm.at[idx], bufs[slot][k], sems[slot])
            if not first:
                drain_store(slot)
            idx = idx_all[0].at[pl.ds(ci * _CHUNK, _CHUNK)]
            pltpu.async_copy(out2_hbm.at[idx], bufs[slot][0], sems[slot])

        def drain(slot):
            for k in range(3):
                pltpu.make_async_copy(
                    out2_hbm.at[pl.ds(0, _CHUNK)], bufs[slot][k],
                    sems[slot]).wait()

        def process(ci, slot):
            b0, b1, b2 = bufs[slot]

            def row_body(r, rc):
                for c in range(d // 16):
                    sl = pl.ds(c * 16, 16)
                    plsc.addupdate(b0.at[r, sl], b1[r, sl] + b2[r, sl])
                return rc

            lax.fori_loop(0, _CHUNK, row_body, 0)
            pltpu.async_copy(
                b0, gsum_hbm.at[pl.ds(base + ci * _CHUNK, _CHUNK)],
                st_sems[slot])

        for s in range(_SLOTS):
            fire(s, s, first=True)

        def group_body(p, carry):
            for s in range(_SLOTS):
                ci = _SLOTS * p + s
                drain(s)
                process(ci, s)

                @pl.when(ci + _SLOTS < n_chunks)
                def _():
                    fire(ci + _SLOTS, s)

            return carry

        lax.fori_loop(0, n_chunks // _SLOTS, group_body, 0)
        for s in range(_SLOTS):
            drain_store(s)

    return sc_gather_sum


def _tc3_body(o1, o2, wc1, wc2, bc, out3):
    out3[...] = (
        jnp.dot(o1[...].astype(jnp.bfloat16), wc1[...],
                preferred_element_type=jnp.float32)
        + jnp.dot(o2[...].astype(jnp.bfloat16), wc2[...],
                  preferred_element_type=jnp.float32)
        + bc[...]
    )


def _tc4_body(o2, g, wa, ba, out4):
    f = ((o2[...] + g[...]) * 0.25).astype(jnp.bfloat16)
    out4[...] = jnp.dot(f, wa[...], preferred_element_type=jnp.float32) + ba[...]


def _split_pairs(per_pair, frac1):
    step = _SLOTS * _CHUNK
    rpw1 = max(step, int(round(per_pair * frac1 / step)) * step)
    return per_pair - rpw1, rpw1


_STAGE_A_FRAC = 0.69
_FRAC1_A = 0.35
_FRAC1_B = 0.40


def kernel(out1, out2, neighbour, W_comb, b_comb, W_agg, b_agg):
    n, d = out2.shape
    dout = b_comb.shape[0]

    step = _SLOTS * _CHUNK
    per_pair = ((n + _NS - 1) // _NS + step - 1) // step * step
    pair_a = max(step, int(round(per_pair * _STAGE_A_FRAC / step)) * step)
    n_a = _NS * pair_a
    pair_b = max(step,
                 ((n - n_a + _NS - 1) // _NS + step - 1) // step * step)
    n_b = _NS * pair_b
    rpw0a, rpw1a = _split_pairs(pair_a, _FRAC1_A)
    rpw0b, rpw1b = _split_pairs(pair_b, _FRAC1_B)
    nbt = jnp.transpose(neighbour.astype(jnp.int32))
    nbt = jnp.pad(
        nbt,
        ((0, 0), (0, n_a + n_b + max(rpw0a, rpw1a, rpw0b, rpw1b) - n)))
    gsum_a = _make_sc_gather_sum(n, d, n_a, rpw0a, rpw1a)(
        out2, nbt[0], nbt[1], nbt[2])
    gsum_b = _make_sc_gather_sum(n, d, n_b, rpw0b, rpw1b, row_off=n_a)(
        out2, nbt[0], nbt[1], nbt[2])

    wcT = jnp.transpose(W_comb[:, :, 0]).astype(jnp.bfloat16)
    wc1 = wcT[:d]
    wc2 = wcT[d:]
    wa = jnp.transpose(W_agg[:, :, 0]).astype(jnp.bfloat16)
    bc = b_comb.reshape(1, dout)
    ba = b_agg.reshape(1, dout)

    blk = 2048
    assert n_a % blk == 0
    row_spec = pl.BlockSpec((blk, d), lambda i: (i, 0))
    out_spec = pl.BlockSpec((blk, dout), lambda i: (i, 0))
    full = lambda s: pl.BlockSpec(s, lambda i: (0, 0))
    out_ty = jax.ShapeDtypeStruct((n, dout), jnp.float32)

    out3 = pl.pallas_call(
        _tc3_body,
        grid=(-(-n // blk),),
        in_specs=[row_spec, row_spec, full((d, dout)), full((d, dout)),
                  full((1, dout))],
        out_specs=out_spec,
        out_shape=out_ty,
    )(out1, out2, wc1, wc2, bc)

    out4_a = pl.pallas_call(
        _tc4_body,
        grid=(n_a // blk,),
        in_specs=[row_spec, row_spec, full((d, dout)), full((1, dout))],
        out_specs=out_spec,
        out_shape=out_ty,
    )(out2, gsum_a, wa, ba)

    nblk_b = -(-n // blk) - n_a // blk
    off_blk = n_a // blk

    def _tc4b_body(prev, o2, g, wa_, ba_, out4):
        del prev
        _tc4_body(o2, g, wa_, ba_, out4)

    out4 = pl.pallas_call(
        _tc4b_body,
        grid=(nblk_b,),
        in_specs=[
            pl.BlockSpec(memory_space=pl.ANY),
            pl.BlockSpec((blk, d), lambda i, o=off_blk: (i + o, 0)),
            pl.BlockSpec((blk, d), lambda i: (i, 0)),
            full((d, dout)), full((1, dout)),
        ],
        out_specs=pl.BlockSpec((blk, dout), lambda i, o=off_blk: (i + o, 0)),
        out_shape=out_ty,
        input_output_aliases={0: 0},
    )(out4_a, out2, gsum_b, wa, ba)
    return (out3, out4)
